# baseline (speedup 1.0000x reference)
import functools

import jax
import jax.numpy as jnp
from jax import lax
from jax.experimental import pallas as pl
from jax.experimental.pallas import tpu as pltpu
from jax.experimental.pallas import tpu_sc as plsc

_NW = 32
_CHUNK = 128
_NBUF = 8


def _flat_gather(bias, idx):
    B = idx.shape[0]
    D = bias.shape[1]
    b_per_w = B // _NW
    n_chunks = b_per_w // _CHUNK
    n_groups = n_chunks // _NBUF
    mesh = plsc.VectorSubcoreMesh(core_axis_name="c", subcore_axis_name="s")

    @functools.partial(
        pl.kernel,
        mesh=mesh,
        out_type=jax.ShapeDtypeStruct((B, 2 * D), jnp.float32),
        scratch_types=(
            [pltpu.VMEM((b_per_w,), jnp.int32)]
            + [pltpu.VMEM((_CHUNK, D), jnp.float32) for _ in range(_NBUF)]
            + [pltpu.SemaphoreType.DMA] * (2 * _NBUF)
        ),
        compiler_params=pltpu.CompilerParams(use_tc_tiling_on_sc=False),
    )
    def k(bias_hbm, idx_hbm, out_hbm, idx_v, *s):
        rows = s[:_NBUF]
        gsem = s[_NBUF:2 * _NBUF]
        wsem = s[2 * _NBUF:3 * _NBUF]
        wid = lax.axis_index("s") * 2 + lax.axis_index("c")
        base = wid * b_per_w
        pltpu.sync_copy(idx_hbm.at[pl.ds(base, b_per_w)], idx_v)

        def fire_gather(j, b):
            pltpu.make_async_copy(
                bias_hbm.at[idx_v.at[pl.ds(j * _CHUNK, _CHUNK)]], rows[b], gsem[b]
            ).start()

        def wait_gather(b):
            pltpu.make_async_copy(bias_hbm.at[pl.ds(0, _CHUNK)], rows[b], gsem[b]).wait()

        def fire_write(j, b):
            pltpu.make_async_copy(
                rows[b],
                out_hbm.at[pl.ds(base + j * _CHUNK, _CHUNK), pl.ds(0, D)],
                wsem[b],
            ).start()

        def wait_write(b):
            pltpu.make_async_copy(
                rows[b], out_hbm.at[pl.ds(base, _CHUNK), pl.ds(0, D)], wsem[b]
            ).wait()

        for b in range(_NBUF):
            fire_gather(b, b)

        def outer(g, carry):
            for b in range(_NBUF):
                j = g * _NBUF + b
                wait_gather(b)
                fire_write(j, b)
                wait_write(b)
                fire_gather(j + _NBUF, b)
            return carry

        lax.fori_loop(0, n_groups - 1, outer, 0)
        for b in range(_NBUF):
            wait_gather(b)
            fire_write((n_groups - 1) * _NBUF + b, b)
            wait_write(b)

    return k(bias, idx)


def kernel(x, bias):
    idx = x.reshape(-1).astype(jnp.int32)
    outp = _flat_gather(bias, idx)
    return outp[:, : bias.shape[1]].reshape(x.shape + (bias.shape[1],))
